# baseline (speedup 1.0000x reference)
import functools

import jax
import jax.numpy as jnp
from jax import lax
from jax.experimental import pallas as pl
from jax.experimental.pallas import tpu as pltpu

_T = 16
_S_CHUNK = 32
_LANE_TILE = 256
_K_IN = 3
_K_OUT = 3
_UNROLL = 8


def _lif_body(x_hbm, y_hbm, mem_ref, res_ref, in_buf, out_buf, in_sem,
              out_sem, *, sc, n_c, B, td, unroll):
    j = pl.program_id(0)
    d0 = j * td

    def in_copy(c, slot):
        return pltpu.make_async_copy(
            x_hbm.at[pl.ds(c * sc, sc), :, pl.ds(d0, td)],
            in_buf.at[slot], in_sem.at[slot])

    def out_copy(c, slot):
        return pltpu.make_async_copy(
            out_buf.at[slot],
            y_hbm.at[pl.ds(c * sc, sc), :, pl.ds(d0, td)],
            out_sem.at[slot])

    for c in range(min(_K_IN, n_c)):
        in_copy(c, c % _K_IN).start()

    one = jnp.float32(1.0)
    zero = jnp.float32(0.0)

    def chunk(c, carry):
        m, r = carry
        islot = lax.rem(c, _K_IN)
        oslot = lax.rem(c, _K_OUT)
        in_copy(c, islot).wait()

        @pl.when(c >= _K_OUT)
        def _():
            out_copy(c - _K_OUT, oslot).wait()

        def step(t, mr):
            m, r = mr
            m = m + in_buf[islot, t]
            cnd = m >= one
            r = 0.5 * r + jnp.where(cnd, one, zero)
            out_buf[oslot, t] = r
            m = jnp.where(cnd, zero, m)
            return m, r

        m, r = lax.fori_loop(0, sc, step, (m, r), unroll=unroll)

        out_copy(c, oslot).start()
        @pl.when(c + _K_IN < n_c)
        def _():
            in_copy(c + _K_IN, lax.rem(c + _K_IN, _K_IN)).start()
        return m, r

    zeros = jnp.zeros((B, td), jnp.float32)
    m, r = lax.fori_loop(0, n_c, chunk, (zeros, zeros))
    mem_ref[...] = m
    res_ref[...] = r

    tail = min(_K_OUT, n_c)
    for i in range(tail):
        c = n_c - tail + i
        out_copy(c, c % _K_OUT).wait()


def kernel(x):
    steps, TB, D = x.shape
    B = TB // _T
    S = steps * _T

    xk = x.reshape(S, B, D)

    td = _LANE_TILE if D % _LANE_TILE == 0 else D
    n_d = D // td
    sc = _S_CHUNK if S % _S_CHUNK == 0 else S
    n_c = S // sc

    body = functools.partial(_lif_body, sc=sc, n_c=n_c, B=B, td=td,
                             unroll=_UNROLL)

    y, mem, res = pl.pallas_call(
        body,
        out_shape=(
            jax.ShapeDtypeStruct((S, B, D), jnp.float32),
            jax.ShapeDtypeStruct((B, D), jnp.float32),
            jax.ShapeDtypeStruct((B, D), jnp.float32),
        ),
        grid=(n_d,),
        in_specs=[pl.BlockSpec(memory_space=pl.ANY)],
        out_specs=(
            pl.BlockSpec(memory_space=pl.ANY),
            pl.BlockSpec((B, td), lambda j: (0, j)),
            pl.BlockSpec((B, td), lambda j: (0, j)),
        ),
        scratch_shapes=[
            pltpu.VMEM((_K_IN, sc, B, td), jnp.float32),
            pltpu.VMEM((_K_OUT, sc, B, td), jnp.float32),
            pltpu.SemaphoreType.DMA((_K_IN,)),
            pltpu.SemaphoreType.DMA((_K_OUT,)),
        ],
        compiler_params=pltpu.CompilerParams(
            dimension_semantics=("parallel",),
            vmem_limit_bytes=64 * 1024 * 1024,
        ),
    )(xk)

    return y.reshape(steps, TB, D), mem, res


# speedup vs baseline: 1.1191x; 1.1191x over previous
import functools

import jax
import jax.numpy as jnp
from jax import lax
from jax.experimental import pallas as pl
from jax.experimental.pallas import tpu as pltpu

_T = 16
_S_CHUNK = 128
_N_PAR = 2
_UNROLL = 8


def _lif_body(x_ref, y_ref, mem_ref, res_ref, *, s_chunk, unroll):
    sc = pl.program_id(1)

    @pl.when(sc == 0)
    def _init():
        mem_ref[...] = jnp.zeros_like(mem_ref)
        res_ref[...] = jnp.zeros_like(res_ref)

    one = jnp.float32(1.0)
    zero = jnp.float32(0.0)

    def step(t, carry):
        m, r = carry
        m = m + x_ref[t]
        c = m >= one
        r = 0.5 * r + jnp.where(c, one, zero)
        y_ref[t] = r
        m = jnp.where(c, zero, m)
        return m, r

    m, r = lax.fori_loop(0, s_chunk, step, (mem_ref[...], res_ref[...]),
                         unroll=unroll)
    mem_ref[...] = m
    res_ref[...] = r


def kernel(x):
    steps, TB, D = x.shape
    B = TB // _T
    S = steps * _T

    xk = x.reshape(S, B, D)

    tb = B // _N_PAR if B % _N_PAR == 0 else B
    n_b = B // tb
    s_chunk = _S_CHUNK if S % _S_CHUNK == 0 else S
    n_s = S // s_chunk

    body = functools.partial(_lif_body, s_chunk=s_chunk, unroll=_UNROLL)

    y, mem, res = pl.pallas_call(
        body,
        out_shape=(
            jax.ShapeDtypeStruct((S, B, D), jnp.float32),
            jax.ShapeDtypeStruct((B, D), jnp.float32),
            jax.ShapeDtypeStruct((B, D), jnp.float32),
        ),
        grid=(n_b, n_s),
        in_specs=[pl.BlockSpec((s_chunk, tb, D), lambda j, s: (s, j, 0))],
        out_specs=(
            pl.BlockSpec((s_chunk, tb, D), lambda j, s: (s, j, 0)),
            pl.BlockSpec((tb, D), lambda j, s: (j, 0)),
            pl.BlockSpec((tb, D), lambda j, s: (j, 0)),
        ),
        compiler_params=pltpu.CompilerParams(
            dimension_semantics=("parallel", "arbitrary"),
            vmem_limit_bytes=64 * 1024 * 1024,
        ),
    )(xk)

    return y.reshape(steps, TB, D), mem, res
